# Initial kernel scaffold; baseline (speedup 1.0000x reference)
import functools

import jax
import jax.numpy as jnp
from jax import lax
from jax.experimental import pallas as pl
from jax.experimental.pallas import tpu as pltpu
from jax.experimental.pallas import tpu_sc as plsc

NC = 2
NS = 16
NW = NC * NS
LANES = 16

CHUNK = 80


def _make_sc_agg(n_nodes, feat, n_chunks, compute_deg):
  rows_per_sub = n_nodes // NS
  zrows = 125
  n_zchunks = rows_per_sub // zrows

  out_type = [jax.ShapeDtypeStruct((NC, n_nodes, feat), jnp.float32)]
  scratch = [
      pltpu.VMEM((n_chunks, CHUNK), jnp.int32),
      pltpu.VMEM((n_chunks, CHUNK), jnp.int32),
      pltpu.VMEM((CHUNK, feat), jnp.float32),
      pltpu.VMEM((zrows, feat), jnp.float32),
      pltpu.VMEM_SHARED((n_nodes, feat), jnp.float32),
      pltpu.SemaphoreType.DMA,
  ]
  if compute_deg:
    out_type.append(jax.ShapeDtypeStruct((NC, n_nodes, LANES), jnp.float32))
    scratch += [
        pltpu.VMEM((CHUNK, LANES), jnp.float32),
        pltpu.VMEM((zrows, LANES), jnp.float32),
        pltpu.VMEM_SHARED((n_nodes, LANES), jnp.float32),
    ]

  mesh = plsc.VectorSubcoreMesh(core_axis_name="c", subcore_axis_name="s")

  def body(h_hbm, src_hbm, dst_hbm, *rest):
    if compute_deg:
      (agg_out, deg_out, src_v, dst_v, rows_v, zero_v, agg_sh, sem,
       ones_v, zdeg_v, deg_sh) = rest
    else:
      (agg_out, src_v, dst_v, rows_v, zero_v, agg_sh, sem) = rest

    c = lax.axis_index("c")
    s = lax.axis_index("s")
    wid = c * NS + s
    row0 = s * rows_per_sub

    zeros16 = jnp.zeros((LANES,), jnp.float32)
    ones16 = jnp.ones((LANES,), jnp.float32)

    def zfill(r, carry):
      for k in range(feat // LANES):
        zero_v[r, pl.ds(k * LANES, LANES)] = zeros16
      if compute_deg:
        zdeg_v[r, pl.ds(0, LANES)] = zeros16
      return carry
    lax.fori_loop(0, zrows, zfill, 0)

    if compute_deg:
      def ofill(r, carry):
        ones_v[r, pl.ds(0, LANES)] = ones16
        return carry
      lax.fori_loop(0, CHUNK, ofill, 0)

    def zcopy(k, carry):
      pltpu.sync_copy(zero_v, agg_sh.at[pl.ds(row0 + k * zrows, zrows)])
      if compute_deg:
        pltpu.sync_copy(zdeg_v, deg_sh.at[pl.ds(row0 + k * zrows, zrows)])
      return carry
    lax.fori_loop(0, n_zchunks, zcopy, 0)

    pltpu.sync_copy(src_hbm.at[wid], src_v)
    pltpu.sync_copy(dst_hbm.at[wid], dst_v)

    plsc.subcore_barrier()

    def chunk_body(j, carry):
      pltpu.async_copy(h_hbm.at[src_v.at[j]], rows_v, sem).wait()
      pltpu.sync_copy(rows_v, agg_sh.at[dst_v.at[j]], add=True)
      if compute_deg:
        pltpu.sync_copy(ones_v, deg_sh.at[dst_v.at[j]], add=True)
      return carry
    lax.fori_loop(0, n_chunks, chunk_body, 0)

    plsc.subcore_barrier()

    pltpu.sync_copy(agg_sh.at[pl.ds(row0, rows_per_sub)],
                    agg_out.at[c, pl.ds(row0, rows_per_sub)])
    if compute_deg:
      pltpu.sync_copy(deg_sh.at[pl.ds(row0, rows_per_sub)],
                      deg_out.at[c, pl.ds(row0, rows_per_sub)])

  return pl.kernel(body, out_type=out_type, mesh=mesh,
                   scratch_types=scratch)


def _proj_body(x_ref, w_ref, b_ref, o_ref):
  y = jnp.dot(x_ref[...], w_ref[...], preferred_element_type=jnp.float32)
  o_ref[...] = jnp.maximum(y + b_ref[...], 0.0)


def _sage_body(do_relu, h_ref, aggA_ref, aggB_ref, degA_ref, degB_ref,
               ex_ref, ws_ref, bs_ref, wn_ref, bn_ref, o_ref):
  h = h_ref[...]
  agg = aggA_ref[...] + aggB_ref[...]
  deg = degA_ref[...][:, 0:1] + degB_ref[...][:, 0:1]
  mean = agg / jnp.maximum(deg, 1.0)
  y = jnp.dot(h, ws_ref[...], preferred_element_type=jnp.float32) + bs_ref[...]
  y = y + jnp.dot(mean, wn_ref[...], preferred_element_type=jnp.float32)
  y = y + bn_ref[...]
  if do_relu:
    y = jnp.maximum(y, 0.0)
  o_ref[...] = y * ex_ref[...]


def _row_blocked(rb, cols):
  return pl.BlockSpec((rb, cols), lambda i: (i, 0))


def _full(shape):
  nd = len(shape)
  return pl.BlockSpec(shape, lambda i: (0,) * nd)


def kernel(attr_features, clustering_coefficient, bidirectional_links_ratio,
           exist_nodes, edge_index, Wp, bp, Ws1, bs1, Wn1, bn1, Ws2, bs2,
           Wn2, bn2):
  n = attr_features.shape[0]
  e = edge_index.shape[1]
  hid = Wp.shape[1]
  out_dim = Ws2.shape[1]

  ept = e // NW
  n_chunks = ept // CHUNK
  assert ept * NW == e and n_chunks * CHUNK == ept

  exist_col = exist_nodes[:, None]
  x0 = jnp.concatenate(
      [attr_features, clustering_coefficient, bidirectional_links_ratio,
       exist_col], axis=1)

  src3 = edge_index[0].reshape(NW, n_chunks, CHUNK)
  dst3 = edge_index[1].reshape(NW, n_chunks, CHUNK)

  rb = 1000
  grid = (n // rb,)
  din = x0.shape[1]

  h0 = pl.pallas_call(
      _proj_body,
      grid=grid,
      in_specs=[_row_blocked(rb, din), _full((din, hid)), _full((1, hid))],
      out_specs=_row_blocked(rb, hid),
      out_shape=jax.ShapeDtypeStruct((n, hid), jnp.float32),
  )(x0, Wp, bp[None, :])

  sc_agg1 = _make_sc_agg(n, hid, n_chunks, compute_deg=True)
  agg1, deg = sc_agg1(h0, src3, dst3)

  def sage_tc(h, agg, dg, Ws, bs, Wn, bn, do_relu, dout):
    return pl.pallas_call(
        functools.partial(_sage_body, do_relu),
        grid=grid,
        in_specs=[_row_blocked(rb, hid), _row_blocked(rb, hid),
                  _row_blocked(rb, hid), _row_blocked(rb, LANES),
                  _row_blocked(rb, LANES), _row_blocked(rb, 1),
                  _full((hid, dout)), _full((1, dout)),
                  _full((hid, dout)), _full((1, dout))],
        out_specs=_row_blocked(rb, dout),
        out_shape=jax.ShapeDtypeStruct((n, dout), jnp.float32),
    )(h, agg[0], agg[1], dg[0], dg[1], exist_col, Ws, bs[None, :],
      Wn, bn[None, :])

  h1 = sage_tc(h0, agg1, deg, Ws1, bs1, Wn1, bn1, True, hid)

  sc_agg2 = _make_sc_agg(n, hid, n_chunks, compute_deg=False)
  (agg2,) = sc_agg2(h1, src3, dst3)

  out = sage_tc(h1, agg2, deg, Ws2, bs2, Wn2, bn2, False, out_dim)
  return out


# speedup vs baseline: 3.9868x; 3.9868x over previous
import functools

import jax
import jax.numpy as jnp
from jax import lax
from jax.experimental import pallas as pl
from jax.experimental.pallas import tpu as pltpu
from jax.experimental.pallas import tpu_sc as plsc

NC = 2
NS = 16
NW = NC * NS
LANES = 16

CHUNK = 80
ZROWS = 8


def _make_sc_agg(n_nodes, feat, n_chunks):
  rows_per_sub = (-(-n_nodes // NS) + 7) // 8 * 8
  n_pad = rows_per_sub * NS
  n_zchunks = rows_per_sub // ZROWS

  mesh = plsc.VectorSubcoreMesh(core_axis_name="c", subcore_axis_name="s")

  def body(h_hbm, src_hbm, dst_hbm, zfeat_hbm, agg_out, src_v, dst_v,
           rows_v, zero_v, agg_sh, sem):
    c = lax.axis_index("c")
    s = lax.axis_index("s")
    wid = c * NS + s
    row0 = s * rows_per_sub

    pltpu.sync_copy(zfeat_hbm, zero_v)

    def zcopy(k, carry):
      pltpu.sync_copy(zero_v, agg_sh.at[pl.ds(row0 + k * ZROWS, ZROWS)])
      return carry
    lax.fori_loop(0, n_zchunks, zcopy, 0)

    plsc.subcore_barrier()

    def chunk_body(j, carry):
      base = wid * n_chunks + j
      pltpu.sync_copy(src_hbm.at[base], src_v)
      pltpu.sync_copy(dst_hbm.at[base], dst_v)
      pltpu.async_copy(h_hbm.at[src_v], rows_v, sem).wait()
      pltpu.sync_copy(rows_v, agg_sh.at[dst_v], add=True)
      return carry
    lax.fori_loop(0, n_chunks, chunk_body, 0)

    plsc.subcore_barrier()

    pltpu.sync_copy(agg_sh.at[pl.ds(row0, rows_per_sub)],
                    agg_out.at[pl.ds(c * n_pad + row0, rows_per_sub)])

  kern = pl.kernel(
      body, mesh=mesh,
      out_type=[jax.ShapeDtypeStruct((NC * n_pad, feat), jnp.float32)],
      scratch_types=[
          pltpu.VMEM((CHUNK,), jnp.int32),
          pltpu.VMEM((CHUNK,), jnp.int32),
          pltpu.VMEM((CHUNK, feat), jnp.float32),
          pltpu.VMEM((ZROWS, feat), jnp.float32),
          pltpu.VMEM_SHARED((n_pad, feat), jnp.float32),
          pltpu.SemaphoreType.DMA,
      ])
  return kern, n_pad


def _make_sc_deg(n_nodes, feat, n_chunks):
  rows_per_sub = (-(-n_nodes // NS) + 7) // 8 * 8
  n_pad = rows_per_sub * NS
  n_zchunks = rows_per_sub // ZROWS

  mesh = plsc.VectorSubcoreMesh(core_axis_name="c", subcore_axis_name="s")

  def body(dst_hbm, zfeat_hbm, ones_hbm, deg_out, dst_v, ones_v, zero_v,
           deg_sh):
    c = lax.axis_index("c")
    s = lax.axis_index("s")
    wid = c * NS + s
    row0 = s * rows_per_sub

    pltpu.sync_copy(zfeat_hbm, zero_v)
    pltpu.sync_copy(ones_hbm, ones_v)

    def zcopy(k, carry):
      pltpu.sync_copy(zero_v, deg_sh.at[pl.ds(row0 + k * ZROWS, ZROWS)])
      return carry
    lax.fori_loop(0, n_zchunks, zcopy, 0)

    plsc.subcore_barrier()

    def chunk_body(j, carry):
      base = wid * n_chunks + j
      pltpu.sync_copy(dst_hbm.at[base], dst_v)
      pltpu.sync_copy(ones_v, deg_sh.at[dst_v], add=True)
      return carry
    lax.fori_loop(0, n_chunks, chunk_body, 0)

    plsc.subcore_barrier()

    pltpu.sync_copy(deg_sh.at[pl.ds(row0, rows_per_sub)],
                    deg_out.at[pl.ds(c * n_pad + row0, rows_per_sub)])

  kern = pl.kernel(
      body, mesh=mesh,
      out_type=[jax.ShapeDtypeStruct((NC * n_pad, feat), jnp.float32)],
      scratch_types=[
          pltpu.VMEM((CHUNK,), jnp.int32),
          pltpu.VMEM((CHUNK, feat), jnp.float32),
          pltpu.VMEM((ZROWS, feat), jnp.float32),
          pltpu.VMEM_SHARED((n_pad, feat), jnp.float32),
      ])
  return kern, n_pad


def _proj_body(x_ref, w_ref, b_ref, o_ref):
  y = jnp.dot(x_ref[...], w_ref[...], preferred_element_type=jnp.float32)
  o_ref[...] = jnp.maximum(y + b_ref[...], 0.0)


def _sage_body(do_relu, h_ref, aggA_ref, aggB_ref, degA_ref, degB_ref,
               ex_ref, ws_ref, bs_ref, wn_ref, bn_ref, o_ref):
  h = h_ref[...]
  agg = aggA_ref[...] + aggB_ref[...]
  deg = degA_ref[...] + degB_ref[...]
  mean = agg / jnp.maximum(deg, 1.0)
  y = jnp.dot(h, ws_ref[...], preferred_element_type=jnp.float32) + bs_ref[...]
  y = y + jnp.dot(mean, wn_ref[...], preferred_element_type=jnp.float32)
  y = y + bn_ref[...]
  if do_relu:
    y = jnp.maximum(y, 0.0)
  o_ref[...] = y * ex_ref[...]


def _row_blocked(rb, cols):
  return pl.BlockSpec((rb, cols), lambda i: (i, 0))


def _full(shape):
  nd = len(shape)
  return pl.BlockSpec(shape, lambda i: (0,) * nd)


def kernel(attr_features, clustering_coefficient, bidirectional_links_ratio,
           exist_nodes, edge_index, Wp, bp, Ws1, bs1, Wn1, bn1, Ws2, bs2,
           Wn2, bn2):
  n = attr_features.shape[0]
  e = edge_index.shape[1]
  hid = Wp.shape[1]
  out_dim = Ws2.shape[1]

  ept = e // NW
  n_chunks = ept // CHUNK
  assert ept * NW == e and n_chunks * CHUNK == ept

  exist_col = exist_nodes[:, None]
  x0 = jnp.concatenate(
      [attr_features, clustering_coefficient, bidirectional_links_ratio,
       exist_col], axis=1)

  src2 = edge_index[0].reshape(NW * n_chunks, CHUNK)
  dst2 = edge_index[1].reshape(NW * n_chunks, CHUNK)
  zfeat = jnp.zeros((ZROWS, hid), jnp.float32)
  ones_t = jnp.ones((CHUNK, hid), jnp.float32)

  rb = 1000
  grid = (n // rb,)
  din = x0.shape[1]

  h0 = pl.pallas_call(
      _proj_body,
      grid=grid,
      in_specs=[_row_blocked(rb, din), _full((din, hid)), _full((1, hid))],
      out_specs=_row_blocked(rb, hid),
      out_shape=jax.ShapeDtypeStruct((n, hid), jnp.float32),
  )(x0, Wp, bp[None, :])

  sc_deg, n_pad = _make_sc_deg(n, hid, n_chunks)
  (degk,) = sc_deg(dst2, zfeat, ones_t)
  degA = degk[:n, 0:1]
  degB = degk[n_pad:n_pad + n, 0:1]

  sc_agg1, _ = _make_sc_agg(n, hid, n_chunks)
  (agg1,) = sc_agg1(h0, src2, dst2, zfeat)

  def sage_tc(h, agg, Ws, bs, Wn, bn, do_relu, dout):
    return pl.pallas_call(
        functools.partial(_sage_body, do_relu),
        grid=grid,
        in_specs=[_row_blocked(rb, hid), _row_blocked(rb, hid),
                  _row_blocked(rb, hid), _row_blocked(rb, 1),
                  _row_blocked(rb, 1), _row_blocked(rb, 1),
                  _full((hid, dout)), _full((1, dout)),
                  _full((hid, dout)), _full((1, dout))],
        out_specs=_row_blocked(rb, dout),
        out_shape=jax.ShapeDtypeStruct((n, dout), jnp.float32),
    )(h, agg[:n], agg[n_pad:n_pad + n], degA, degB, exist_col, Ws,
      bs[None, :], Wn, bn[None, :])

  h1 = sage_tc(h0, agg1, Ws1, bs1, Wn1, bn1, True, hid)

  sc_agg2, _ = _make_sc_agg(n, hid, n_chunks)
  (agg2,) = sc_agg2(h1, src2, dst2, zfeat)

  out = sage_tc(h1, agg2, Ws2, bs2, Wn2, bn2, False, out_dim)
  return out


# speedup vs baseline: 5.5750x; 1.3984x over previous
import functools

import jax
import jax.numpy as jnp
from jax import lax
from jax.experimental import pallas as pl
from jax.experimental.pallas import tpu as pltpu
from jax.experimental.pallas import tpu_sc as plsc

NC = 2
NS = 16
NW = NC * NS
LANES = 16

CHUNK = 80
ZROWS = 8


def _make_sc_agg(n_nodes, feat, n_chunks):
  rows_per_sub = (-(-n_nodes // NS) + 7) // 8 * 8
  n_pad = rows_per_sub * NS
  n_zchunks = rows_per_sub // ZROWS

  grp = 4
  n_grps = n_chunks // grp
  tail = n_chunks - n_grps * grp

  mesh = plsc.VectorSubcoreMesh(core_axis_name="c", subcore_axis_name="s")

  def body(h_hbm, src_hbm, dst_hbm, zfeat_hbm, agg_out, *rest):
    src_vs = rest[0:grp]
    dst_vs = rest[grp:2 * grp]
    rows_vs = rest[2 * grp:3 * grp]
    zero_v = rest[3 * grp]
    agg_sh = rest[3 * grp + 1]
    sems = rest[3 * grp + 2:3 * grp + 2 + grp]

    c = lax.axis_index("c")
    s = lax.axis_index("s")
    wid = c * NS + s
    row0 = s * rows_per_sub

    pltpu.sync_copy(zfeat_hbm, zero_v)

    def zcopy(k, carry):
      pltpu.sync_copy(zero_v, agg_sh.at[pl.ds(row0 + k * ZROWS, ZROWS)])
      return carry
    lax.fori_loop(0, n_zchunks, zcopy, 0)

    plsc.subcore_barrier()

    def group_body(g, carry):
      base0 = wid * n_chunks + g * grp
      copies = []
      for j in range(grp):
        pltpu.sync_copy(src_hbm.at[base0 + j], src_vs[j])
        pltpu.sync_copy(dst_hbm.at[base0 + j], dst_vs[j])
        copies.append(pltpu.async_copy(h_hbm.at[src_vs[j]], rows_vs[j],
                                       sems[j]))
      for j in range(grp):
        copies[j].wait()
        pltpu.sync_copy(rows_vs[j], agg_sh.at[dst_vs[j]], add=True)
      return carry
    lax.fori_loop(0, n_grps, group_body, 0)

    for t in range(tail):
      base = wid * n_chunks + n_grps * grp + t
      pltpu.sync_copy(src_hbm.at[base], src_vs[t])
      pltpu.sync_copy(dst_hbm.at[base], dst_vs[t])
      pltpu.async_copy(h_hbm.at[src_vs[t]], rows_vs[t], sems[t]).wait()
      pltpu.sync_copy(rows_vs[t], agg_sh.at[dst_vs[t]], add=True)

    plsc.subcore_barrier()

    pltpu.sync_copy(agg_sh.at[pl.ds(row0, rows_per_sub)],
                    agg_out.at[pl.ds(c * n_pad + row0, rows_per_sub)])

  kern = pl.kernel(
      body, mesh=mesh,
      out_type=[jax.ShapeDtypeStruct((NC * n_pad, feat), jnp.float32)],
      scratch_types=(
          [pltpu.VMEM((CHUNK,), jnp.int32)] * grp +
          [pltpu.VMEM((CHUNK,), jnp.int32)] * grp +
          [pltpu.VMEM((CHUNK, feat), jnp.float32)] * grp
          + [
              pltpu.VMEM((ZROWS, feat), jnp.float32),
              pltpu.VMEM_SHARED((n_pad, feat), jnp.float32),
          ] + [pltpu.SemaphoreType.DMA] * grp))
  return kern, n_pad


def _make_sc_deg(n_nodes, feat, n_chunks):
  rows_per_sub = (-(-n_nodes // NS) + 7) // 8 * 8
  n_pad = rows_per_sub * NS
  n_zchunks = rows_per_sub // ZROWS

  mesh = plsc.VectorSubcoreMesh(core_axis_name="c", subcore_axis_name="s")

  def body(dst_hbm, zfeat_hbm, ones_hbm, deg_out, dst_v, ones_v, zero_v,
           deg_sh):
    c = lax.axis_index("c")
    s = lax.axis_index("s")
    wid = c * NS + s
    row0 = s * rows_per_sub

    pltpu.sync_copy(zfeat_hbm, zero_v)
    pltpu.sync_copy(ones_hbm, ones_v)

    def zcopy(k, carry):
      pltpu.sync_copy(zero_v, deg_sh.at[pl.ds(row0 + k * ZROWS, ZROWS)])
      return carry
    lax.fori_loop(0, n_zchunks, zcopy, 0)

    plsc.subcore_barrier()

    def chunk_body(j, carry):
      base = wid * n_chunks + j
      pltpu.sync_copy(dst_hbm.at[base], dst_v)
      pltpu.sync_copy(ones_v, deg_sh.at[dst_v], add=True)
      return carry
    lax.fori_loop(0, n_chunks, chunk_body, 0)

    plsc.subcore_barrier()

    pltpu.sync_copy(deg_sh.at[pl.ds(row0, rows_per_sub)],
                    deg_out.at[pl.ds(c * n_pad + row0, rows_per_sub)])

  kern = pl.kernel(
      body, mesh=mesh,
      out_type=[jax.ShapeDtypeStruct((NC * n_pad, feat), jnp.float32)],
      scratch_types=[
          pltpu.VMEM((CHUNK,), jnp.int32),
          pltpu.VMEM((CHUNK, feat), jnp.float32),
          pltpu.VMEM((ZROWS, feat), jnp.float32),
          pltpu.VMEM_SHARED((n_pad, feat), jnp.float32),
      ])
  return kern, n_pad


def _proj_body(x_ref, w_ref, b_ref, o_ref):
  y = jnp.dot(x_ref[...], w_ref[...], preferred_element_type=jnp.float32)
  o_ref[...] = jnp.maximum(y + b_ref[...], 0.0)


def _sage_body(do_relu, h_ref, aggA_ref, aggB_ref, degA_ref, degB_ref,
               ex_ref, ws_ref, bs_ref, wn_ref, bn_ref, o_ref):
  h = h_ref[...]
  agg = aggA_ref[...] + aggB_ref[...]
  deg = degA_ref[...] + degB_ref[...]
  mean = agg / jnp.maximum(deg, 1.0)
  y = jnp.dot(h, ws_ref[...], preferred_element_type=jnp.float32) + bs_ref[...]
  y = y + jnp.dot(mean, wn_ref[...], preferred_element_type=jnp.float32)
  y = y + bn_ref[...]
  if do_relu:
    y = jnp.maximum(y, 0.0)
  o_ref[...] = y * ex_ref[...]


def _row_blocked(rb, cols):
  return pl.BlockSpec((rb, cols), lambda i: (i, 0))


def _full(shape):
  nd = len(shape)
  return pl.BlockSpec(shape, lambda i: (0,) * nd)


def kernel(attr_features, clustering_coefficient, bidirectional_links_ratio,
           exist_nodes, edge_index, Wp, bp, Ws1, bs1, Wn1, bn1, Ws2, bs2,
           Wn2, bn2):
  n = attr_features.shape[0]
  e = edge_index.shape[1]
  hid = Wp.shape[1]
  out_dim = Ws2.shape[1]

  ept = e // NW
  n_chunks = ept // CHUNK
  assert ept * NW == e and n_chunks * CHUNK == ept

  exist_col = exist_nodes[:, None]
  x0 = jnp.concatenate(
      [attr_features, clustering_coefficient, bidirectional_links_ratio,
       exist_col], axis=1)

  src2 = edge_index[0].reshape(NW * n_chunks, CHUNK)
  dst2 = edge_index[1].reshape(NW * n_chunks, CHUNK)
  zfeat = jnp.zeros((ZROWS, hid), jnp.float32)
  ones_t = jnp.ones((CHUNK, hid), jnp.float32)

  rb = 1000
  grid = (n // rb,)
  din = x0.shape[1]

  h0 = pl.pallas_call(
      _proj_body,
      grid=grid,
      in_specs=[_row_blocked(rb, din), _full((din, hid)), _full((1, hid))],
      out_specs=_row_blocked(rb, hid),
      out_shape=jax.ShapeDtypeStruct((n, hid), jnp.float32),
  )(x0, Wp, bp[None, :])

  sc_deg, n_pad = _make_sc_deg(n, hid, n_chunks)
  (degk,) = sc_deg(dst2, zfeat, ones_t)
  degA = degk[:n, 0:1]
  degB = degk[n_pad:n_pad + n, 0:1]

  sc_agg1, _ = _make_sc_agg(n, hid, n_chunks)
  (agg1,) = sc_agg1(h0, src2, dst2, zfeat)

  def sage_tc(h, agg, Ws, bs, Wn, bn, do_relu, dout):
    return pl.pallas_call(
        functools.partial(_sage_body, do_relu),
        grid=grid,
        in_specs=[_row_blocked(rb, hid), _row_blocked(rb, hid),
                  _row_blocked(rb, hid), _row_blocked(rb, 1),
                  _row_blocked(rb, 1), _row_blocked(rb, 1),
                  _full((hid, dout)), _full((1, dout)),
                  _full((hid, dout)), _full((1, dout))],
        out_specs=_row_blocked(rb, dout),
        out_shape=jax.ShapeDtypeStruct((n, dout), jnp.float32),
    )(h, agg[:n], agg[n_pad:n_pad + n], degA, degB, exist_col, Ws,
      bs[None, :], Wn, bn[None, :])

  h1 = sage_tc(h0, agg1, Ws1, bs1, Wn1, bn1, True, hid)

  sc_agg2, _ = _make_sc_agg(n, hid, n_chunks)
  (agg2,) = sc_agg2(h1, src2, dst2, zfeat)

  out = sage_tc(h1, agg2, Ws2, bs2, Wn2, bn2, False, out_dim)
  return out


# speedup vs baseline: 5.5897x; 1.0026x over previous
import functools

import jax
import jax.numpy as jnp
from jax import lax
from jax.experimental import pallas as pl
from jax.experimental.pallas import tpu as pltpu
from jax.experimental.pallas import tpu_sc as plsc

NC = 2
NS = 16
NW = NC * NS
LANES = 16

CHUNK = 80
ZROWS = 8


def _make_sc_agg(n_nodes, feat, n_chunks):
  rows_per_sub = (-(-n_nodes // NS) + 7) // 8 * 8
  n_pad = rows_per_sub * NS
  n_zchunks = rows_per_sub // ZROWS

  grp = 4
  n_grps = n_chunks // grp
  tail = n_chunks - n_grps * grp

  mesh = plsc.VectorSubcoreMesh(core_axis_name="c", subcore_axis_name="s")

  def body(h_hbm, src_hbm, dst_hbm, zfeat_hbm, agg_out, *rest):
    src_vs = rest[0:grp]
    dst_vs = rest[grp:2 * grp]
    rows_vs = rest[2 * grp:3 * grp]
    zero_v = rest[3 * grp]
    agg_sh = rest[3 * grp + 1]
    sems = rest[3 * grp + 2:3 * grp + 2 + grp]

    c = lax.axis_index("c")
    s = lax.axis_index("s")
    wid = c * NS + s
    row0 = s * rows_per_sub

    pltpu.sync_copy(zfeat_hbm, zero_v)

    def zcopy(k, carry):
      pltpu.sync_copy(zero_v, agg_sh.at[pl.ds(row0 + k * ZROWS, ZROWS)])
      return carry
    lax.fori_loop(0, n_zchunks, zcopy, 0)

    plsc.subcore_barrier()

    def fire(j, base):
      pltpu.sync_copy(src_hbm.at[base], src_vs[j])
      pltpu.sync_copy(dst_hbm.at[base], dst_vs[j])
      pltpu.async_copy(h_hbm.at[src_vs[j]], rows_vs[j], sems[j])

    def drain(j):
      pltpu.make_async_copy(h_hbm.at[src_vs[j]], rows_vs[j], sems[j]).wait()
      pltpu.sync_copy(rows_vs[j], agg_sh.at[dst_vs[j]], add=True)

    for j in range(grp):
      fire(j, wid * n_chunks + j)

    def group_body(g, carry):
      base0 = wid * n_chunks + g * grp
      for j in range(grp):
        drain(j)
        fire(j, base0 + j)
      return carry
    lax.fori_loop(1, n_grps, group_body, 0)

    for j in range(grp):
      drain(j)
    for t in range(tail):
      fire(t, wid * n_chunks + n_grps * grp + t)
      drain(t)

    plsc.subcore_barrier()

    pltpu.sync_copy(agg_sh.at[pl.ds(row0, rows_per_sub)],
                    agg_out.at[pl.ds(c * n_pad + row0, rows_per_sub)])

  kern = pl.kernel(
      body, mesh=mesh,
      out_type=[jax.ShapeDtypeStruct((NC * n_pad, feat), jnp.float32)],
      scratch_types=(
          [pltpu.VMEM((CHUNK,), jnp.int32)] * grp +
          [pltpu.VMEM((CHUNK,), jnp.int32)] * grp +
          [pltpu.VMEM((CHUNK, feat), jnp.float32)] * grp
          + [
              pltpu.VMEM((ZROWS, feat), jnp.float32),
              pltpu.VMEM_SHARED((n_pad, feat), jnp.float32),
          ] + [pltpu.SemaphoreType.DMA] * grp))
  return kern, n_pad


def _make_sc_deg(n_nodes, feat, n_chunks):
  rows_per_sub = (-(-n_nodes // NS) + 7) // 8 * 8
  n_pad = rows_per_sub * NS
  n_zchunks = rows_per_sub // ZROWS

  mesh = plsc.VectorSubcoreMesh(core_axis_name="c", subcore_axis_name="s")

  def body(dst_hbm, zfeat_hbm, ones_hbm, deg_out, dst_v, ones_v, zero_v,
           deg_sh):
    c = lax.axis_index("c")
    s = lax.axis_index("s")
    wid = c * NS + s
    row0 = s * rows_per_sub

    pltpu.sync_copy(zfeat_hbm, zero_v)
    pltpu.sync_copy(ones_hbm, ones_v)

    def zcopy(k, carry):
      pltpu.sync_copy(zero_v, deg_sh.at[pl.ds(row0 + k * ZROWS, ZROWS)])
      return carry
    lax.fori_loop(0, n_zchunks, zcopy, 0)

    plsc.subcore_barrier()

    def chunk_body(j, carry):
      base = wid * n_chunks + j
      pltpu.sync_copy(dst_hbm.at[base], dst_v)
      pltpu.sync_copy(ones_v, deg_sh.at[dst_v], add=True)
      return carry
    lax.fori_loop(0, n_chunks, chunk_body, 0)

    plsc.subcore_barrier()

    pltpu.sync_copy(deg_sh.at[pl.ds(row0, rows_per_sub)],
                    deg_out.at[pl.ds(c * n_pad + row0, rows_per_sub)])

  kern = pl.kernel(
      body, mesh=mesh,
      out_type=[jax.ShapeDtypeStruct((NC * n_pad, feat), jnp.float32)],
      scratch_types=[
          pltpu.VMEM((CHUNK,), jnp.int32),
          pltpu.VMEM((CHUNK, feat), jnp.float32),
          pltpu.VMEM((ZROWS, feat), jnp.float32),
          pltpu.VMEM_SHARED((n_pad, feat), jnp.float32),
      ])
  return kern, n_pad


def _proj_body(x_ref, w_ref, b_ref, o_ref):
  y = jnp.dot(x_ref[...], w_ref[...], preferred_element_type=jnp.float32)
  o_ref[...] = jnp.maximum(y + b_ref[...], 0.0)


def _sage_body(do_relu, h_ref, aggA_ref, aggB_ref, degA_ref, degB_ref,
               ex_ref, ws_ref, bs_ref, wn_ref, bn_ref, o_ref):
  h = h_ref[...]
  agg = aggA_ref[...] + aggB_ref[...]
  deg = degA_ref[...] + degB_ref[...]
  mean = agg / jnp.maximum(deg, 1.0)
  y = jnp.dot(h, ws_ref[...], preferred_element_type=jnp.float32) + bs_ref[...]
  y = y + jnp.dot(mean, wn_ref[...], preferred_element_type=jnp.float32)
  y = y + bn_ref[...]
  if do_relu:
    y = jnp.maximum(y, 0.0)
  o_ref[...] = y * ex_ref[...]


def _row_blocked(rb, cols):
  return pl.BlockSpec((rb, cols), lambda i: (i, 0))


def _full(shape):
  nd = len(shape)
  return pl.BlockSpec(shape, lambda i: (0,) * nd)


def kernel(attr_features, clustering_coefficient, bidirectional_links_ratio,
           exist_nodes, edge_index, Wp, bp, Ws1, bs1, Wn1, bn1, Ws2, bs2,
           Wn2, bn2):
  n = attr_features.shape[0]
  e = edge_index.shape[1]
  hid = Wp.shape[1]
  out_dim = Ws2.shape[1]

  ept = e // NW
  n_chunks = ept // CHUNK
  assert ept * NW == e and n_chunks * CHUNK == ept

  exist_col = exist_nodes[:, None]
  x0 = jnp.concatenate(
      [attr_features, clustering_coefficient, bidirectional_links_ratio,
       exist_col], axis=1)

  src2 = edge_index[0].reshape(NW * n_chunks, CHUNK)
  dst2 = edge_index[1].reshape(NW * n_chunks, CHUNK)
  zfeat = jnp.zeros((ZROWS, hid), jnp.float32)
  ones_t = jnp.ones((CHUNK, hid), jnp.float32)

  rb = 1000
  grid = (n // rb,)
  din = x0.shape[1]

  h0 = pl.pallas_call(
      _proj_body,
      grid=grid,
      in_specs=[_row_blocked(rb, din), _full((din, hid)), _full((1, hid))],
      out_specs=_row_blocked(rb, hid),
      out_shape=jax.ShapeDtypeStruct((n, hid), jnp.float32),
  )(x0, Wp, bp[None, :])

  sc_deg, n_pad = _make_sc_deg(n, hid, n_chunks)
  (degk,) = sc_deg(dst2, zfeat, ones_t)
  degA = degk[:n, 0:1]
  degB = degk[n_pad:n_pad + n, 0:1]

  sc_agg1, _ = _make_sc_agg(n, hid, n_chunks)
  (agg1,) = sc_agg1(h0, src2, dst2, zfeat)

  def sage_tc(h, agg, Ws, bs, Wn, bn, do_relu, dout):
    return pl.pallas_call(
        functools.partial(_sage_body, do_relu),
        grid=grid,
        in_specs=[_row_blocked(rb, hid), _row_blocked(rb, hid),
                  _row_blocked(rb, hid), _row_blocked(rb, 1),
                  _row_blocked(rb, 1), _row_blocked(rb, 1),
                  _full((hid, dout)), _full((1, dout)),
                  _full((hid, dout)), _full((1, dout))],
        out_specs=_row_blocked(rb, dout),
        out_shape=jax.ShapeDtypeStruct((n, dout), jnp.float32),
    )(h, agg[:n], agg[n_pad:n_pad + n], degA, degB, exist_col, Ws,
      bs[None, :], Wn, bn[None, :])

  h1 = sage_tc(h0, agg1, Ws1, bs1, Wn1, bn1, True, hid)

  sc_agg2, _ = _make_sc_agg(n, hid, n_chunks)
  (agg2,) = sc_agg2(h1, src2, dst2, zfeat)

  out = sage_tc(h1, agg2, Ws2, bs2, Wn2, bn2, False, out_dim)
  return out
